# baseline (speedup 1.0000x reference)
import jax
import jax.numpy as jnp
import numpy as np
from jax import lax
from jax.experimental import pallas as pl
from jax.experimental.pallas import tpu as pltpu
from jax.experimental.pallas import tpu_sc as plsc

N = 10000
E = 320000
H = 128
ED = 4
G = 64

DX = 144
EB = 128
NBLK = E // EB
NCH = 2
NBLKC = NBLK // NCH
EC = NBLKC * EB
NC = 2
NS = 16
NW = NC * NS
NBT = -(-NBLKC // NW)
NPAIR = (NBT + 1) // 2
NPT = N // NS
NPC = 125

_f32 = jnp.float32


def _silu(v):
    return v * jax.nn.sigmoid(v)


def _tile_blocks(wid):
    q, r = NBLKC // NW, NBLKC % NW
    start = q * wid + jnp.minimum(wid, r)
    nb = jnp.where(wid < r, q + 1, q)
    return start, nb


def _load_idx(src_hbm, start, nb, dst):
    @pl.when(nb == NBT)
    def _():
        pltpu.sync_copy(src_hbm.at[pl.ds(start, NBT)], dst)

    @pl.when(nb < NBT)
    def _():
        pltpu.sync_copy(src_hbm.at[pl.ds(start, NBT - 1)],
                        dst.at[pl.ds(0, NBT - 1)])


def _pre_body(h_ref, xp_ref, w1a_ref, w1b_ref, be1_ref, a_ref, b_ref):
    hb = h_ref[...]
    a = jnp.dot(hb, w1a_ref[...], preferred_element_type=_f32) + be1_ref[...]
    b = jnp.dot(hb, w1b_ref[...], preferred_element_type=_f32)
    a_ref[:, :H] = a
    a_ref[:, H:] = xp_ref[...]
    b_ref[:, :H] = b
    b_ref[:, H:] = xp_ref[...]


def _precompute(h, xp, w1a, w1b, be1):
    R = 2000
    grid = N // R
    return pl.pallas_call(
        _pre_body,
        grid=(grid,),
        in_specs=[
            pl.BlockSpec((R, H), lambda i: (i, 0)),
            pl.BlockSpec((R, DX - H), lambda i: (i, 0)),
            pl.BlockSpec((H, H), lambda i: (0, 0)),
            pl.BlockSpec((H, H), lambda i: (0, 0)),
            pl.BlockSpec((1, H), lambda i: (0, 0)),
        ],
        out_specs=[
            pl.BlockSpec((R, DX), lambda i: (i, 0)),
            pl.BlockSpec((R, DX), lambda i: (i, 0)),
        ],
        out_shape=[
            jax.ShapeDtypeStruct((N, DX), _f32),
            jax.ShapeDtypeStruct((N, DX), _f32),
        ],
    )(h, xp, w1a, w1b, be1)


def _sc_edge_body(a_hbm, b_hbm, row2_hbm, col2_hbm, w1r_hbm, t_hbm,
                  idxr_v, idxc_v, ar0, br0, ar1, br1, tb0, tb1, w1r_v,
                  sa0, sb0, sa1, sb1, st0, st1):
    wid = lax.axis_index("s") * NC + lax.axis_index("c")
    start, nb = _tile_blocks(wid)
    pltpu.sync_copy(w1r_hbm, w1r_v)
    _load_idx(row2_hbm, start, nb, idxr_v)
    _load_idx(col2_hbm, start, nb, idxc_v)

    def fetch(k, ar, br, sa, sb):
        pltpu.async_copy(a_hbm.at[idxr_v.at[k]], ar, sa)
        pltpu.async_copy(b_hbm.at[idxc_v.at[k]], br, sb)

    def compute(ar, br, tb):
        @plsc.parallel_loop(0, EB, unroll=2)
        def _edges(e):
            xa = ar[e, pl.ds(H, 16)]
            xb = br[e, pl.ds(H, 16)]
            d = xa - xb
            dd = d * d
            r = dd[0] + dd[1] + dd[2]
            for p in range(H // 32):
                sa = pl.ds(32 * p, 16)
                sb = pl.ds(32 * p + 16, 16)
                va = ar[e, sa] + br[e, sa] + r * w1r_v[sa]
                vb = ar[e, sb] + br[e, sb] + r * w1r_v[sb]
                tb[e, pl.ds(32 * p, 32)] = plsc.pack(
                    va, vb, format=plsc.PackFormat.INTERLEAVED)

    def phase(j, k, ar, br, tb, sa, sb, st):
        @pl.when(k < nb)
        def _():
            pltpu.make_async_copy(a_hbm.at[idxr_v.at[0]], ar, sa).wait()
            pltpu.make_async_copy(b_hbm.at[idxc_v.at[0]], br, sb).wait()

            @pl.when(j > 0)
            def _():
                pltpu.make_async_copy(tb, t_hbm.at[pl.ds(0, EB)], st).wait()

            compute(ar, br, tb)
            pltpu.async_copy(tb, t_hbm.at[pl.ds((start + k) * EB, EB)], st)

        @pl.when(k + 2 < nb)
        def _():
            fetch(k + 2, ar, br, sa, sb)

    fetch(0, ar0, br0, sa0, sb0)
    fetch(1, ar1, br1, sa1, sb1)

    @pl.loop(0, NPAIR)
    def _pairs(j):
        phase(j, 2 * j, ar0, br0, tb0, sa0, sb0, st0)
        phase(j, 2 * j + 1, ar1, br1, tb1, sa1, sb1, st1)

    pltpu.make_async_copy(tb0, t_hbm.at[pl.ds(0, EB)], st0).wait()
    pltpu.make_async_copy(tb1, t_hbm.at[pl.ds(0, EB)], st1).wait()


def _sc_edge(a_ext, b_ext, row2c, col2c, w1r):
    mesh = plsc.VectorSubcoreMesh(core_axis_name="c", subcore_axis_name="s")
    fn = pl.kernel(
        _sc_edge_body,
        out_type=jax.ShapeDtypeStruct((EC, H), jnp.bfloat16),
        mesh=mesh,
        compiler_params=pltpu.CompilerParams(use_tc_tiling_on_sc=False,
                                             needs_layout_passes=False),
        scratch_types=[
            pltpu.VMEM((NBT, EB), jnp.int32),
            pltpu.VMEM((NBT, EB), jnp.int32),
            pltpu.VMEM((EB, DX), _f32),
            pltpu.VMEM((EB, DX), _f32),
            pltpu.VMEM((EB, DX), _f32),
            pltpu.VMEM((EB, DX), _f32),
            pltpu.VMEM((EB, H), jnp.bfloat16),
            pltpu.VMEM((EB, H), jnp.bfloat16),
            pltpu.VMEM((H,), _f32),
            pltpu.SemaphoreType.DMA,
            pltpu.SemaphoreType.DMA,
            pltpu.SemaphoreType.DMA,
            pltpu.SemaphoreType.DMA,
            pltpu.SemaphoreType.DMA,
            pltpu.SemaphoreType.DMA,
        ],
    )
    return fn(a_ext, b_ext, row2c, col2c, w1r)


def _mlp2_body(t_ref, eat_ref, w1d_ref, we2_ref, be2_ref, o_ref):
    ea_term = lax.dot_general(eat_ref[...], w1d_ref[...],
                              (((0,), (0,)), ((), ())),
                              preferred_element_type=_f32)
    ef = _silu(t_ref[...].astype(_f32) + ea_term)
    o_ref[...] = _silu(jnp.dot(ef, we2_ref[...],
                               preferred_element_type=_f32) + be2_ref[...])


def _mlp2(t, eat, w1d, we2, be2):
    R = 1280
    grid = EC // R
    return pl.pallas_call(
        _mlp2_body,
        grid=(grid,),
        in_specs=[
            pl.BlockSpec((R, H), lambda i: (i, 0)),
            pl.BlockSpec((ED, R), lambda i: (0, i)),
            pl.BlockSpec((ED, H), lambda i: (0, 0)),
            pl.BlockSpec((H, H), lambda i: (0, 0)),
            pl.BlockSpec((1, H), lambda i: (0, 0)),
        ],
        out_specs=pl.BlockSpec((R, H), lambda i: (i, 0)),
        out_shape=jax.ShapeDtypeStruct((EC, H), _f32),
    )(t, eat, w1d, we2, be2)


def _sc_scatter_body(ef2_hbm, row2_hbm, agg_hbm,
                     idx_v, fb0, fb1, acc_sh, sf0, sf1, ss0, ss1):
    cid = lax.axis_index("c")
    sid = lax.axis_index("s")
    wid = sid * NC + cid
    start, nb = _tile_blocks(wid)
    _load_idx(row2_hbm, start, nb, idx_v)

    zero16 = jnp.zeros((16,), _f32)

    @pl.loop(0, NPC)
    def _zrow(e):
        for j in range(H // 16):
            fb0[e, pl.ds(16 * j, 16)] = zero16
    for i in range(NPT // NPC):
        pltpu.sync_copy(fb0.at[pl.ds(0, NPC)],
                        acc_sh.at[pl.ds(sid * NPT + i * NPC, NPC)])
    plsc.subcore_barrier()

    def phase_fetch(j, k, fb, sf, ss):
        @pl.when(k < nb)
        def _():
            @pl.when(j > 0)
            def _():
                pltpu.make_async_copy(fb, acc_sh.at[idx_v.at[0]], ss).wait()
            pltpu.async_copy(ef2_hbm.at[pl.ds((start + k) * EB, EB)], fb, sf)

    def phase_scatter(k, fb, sf, ss):
        @pl.when(k < nb)
        def _():
            pltpu.make_async_copy(
                ef2_hbm.at[pl.ds(0, EB)], fb, sf).wait()
            pltpu.async_copy(fb, acc_sh.at[idx_v.at[k]], ss, add=True)

    @pl.loop(0, NPAIR)
    def _pairs(j):
        phase_fetch(j, 2 * j, fb0, sf0, ss0)
        phase_fetch(j, 2 * j + 1, fb1, sf1, ss1)
        phase_scatter(2 * j, fb0, sf0, ss0)
        phase_scatter(2 * j + 1, fb1, sf1, ss1)

    pltpu.make_async_copy(fb0, acc_sh.at[idx_v.at[0]], ss0).wait()
    pltpu.make_async_copy(fb1, acc_sh.at[idx_v.at[0]], ss1).wait()

    plsc.subcore_barrier()
    for i in range(NPT // NPC):
        base = sid * NPT + i * NPC
        pltpu.sync_copy(acc_sh.at[pl.ds(base, NPC)], fb0.at[pl.ds(0, NPC)])
        pltpu.sync_copy(fb0.at[pl.ds(0, NPC)], agg_hbm.at[cid, pl.ds(base, NPC)])


def _sc_scatter(ef2c, row2c):
    mesh = plsc.VectorSubcoreMesh(core_axis_name="c", subcore_axis_name="s")
    fn = pl.kernel(
        _sc_scatter_body,
        out_type=jax.ShapeDtypeStruct((NC, N, H), _f32),
        mesh=mesh,
        compiler_params=pltpu.CompilerParams(use_tc_tiling_on_sc=False),
        scratch_types=[
            pltpu.VMEM((NBT, EB), jnp.int32),
            pltpu.VMEM((EB, H), _f32),
            pltpu.VMEM((EB, H), _f32),
            pltpu.VMEM_SHARED((N, H), _f32),
            pltpu.SemaphoreType.DMA,
            pltpu.SemaphoreType.DMA,
            pltpu.SemaphoreType.DMA,
            pltpu.SemaphoreType.DMA,
        ],
    )
    return fn(ef2c, row2c)


def _node_body(h_ref, agga_ref, aggb_ref, wn1a_ref, wn1b_ref, bn1_ref,
               wn2_ref, bn2_ref, batch_ref, p_ref, sums, cnt):
    i = pl.program_id(0)

    @pl.when(i == 0)
    def _init():
        sums[...] = jnp.zeros((G, H), _f32)
        cnt[...] = jnp.zeros((G, H), _f32)

    hb = h_ref[...]
    agg = (agga_ref[0] + agga_ref[1]) + (aggb_ref[0] + aggb_ref[1])
    pre = (jnp.dot(hb, wn1a_ref[...], preferred_element_type=_f32)
           + jnp.dot(agg, wn1b_ref[...], preferred_element_type=_f32)
           + bn1_ref[...])
    o = _silu(pre)
    o2 = jnp.dot(o, wn2_ref[...], preferred_element_type=_f32) + bn2_ref[...]
    h2 = hb + o2

    bvec = batch_ref[0, 0, :]
    R = h2.shape[0]
    oh = (bvec[:, None] == lax.broadcasted_iota(jnp.int32, (R, G), 1))
    ohf = oh.astype(_f32)
    sums[...] += lax.dot_general(ohf, h2, (((0,), (0,)), ((), ())),
                                 preferred_element_type=_f32)
    cnt[...] += jnp.sum(ohf, axis=0)[:, None]

    @pl.when(i == pl.num_programs(0) - 1)
    def _fin():
        p_ref[...] = sums[...] / jnp.maximum(cnt[...], 1.0)


def _node_pool(h, agga, aggb, wn1a, wn1b, bn1, wn2, bn2, batch3):
    R = 1000
    grid = N // R
    return pl.pallas_call(
        _node_body,
        grid=(grid,),
        in_specs=[
            pl.BlockSpec((R, H), lambda i: (i, 0)),
            pl.BlockSpec((NC, R, H), lambda i: (0, i, 0)),
            pl.BlockSpec((NC, R, H), lambda i: (0, i, 0)),
            pl.BlockSpec((H, H), lambda i: (0, 0)),
            pl.BlockSpec((H, H), lambda i: (0, 0)),
            pl.BlockSpec((1, H), lambda i: (0, 0)),
            pl.BlockSpec((H, H), lambda i: (0, 0)),
            pl.BlockSpec((1, H), lambda i: (0, 0)),
            pl.BlockSpec((1, 1, R), lambda i: (i, 0, 0)),
        ],
        out_specs=pl.BlockSpec((G, H), lambda i: (0, 0)),
        out_shape=jax.ShapeDtypeStruct((G, H), _f32),
        scratch_shapes=[
            pltpu.VMEM((G, H), _f32),
            pltpu.VMEM((G, H), _f32),
        ],
    )(h, agga, aggb, wn1a, wn1b, bn1, wn2, bn2, batch3)


def kernel(h, edge_index, x, edge_attr, batch, We1, be1, We2, be2,
           Wn1, bn1, Wn2, bn2):
    row2 = edge_index[0].reshape(NBLK, EB)
    col2 = edge_index[1].reshape(NBLK, EB)
    xp = jnp.zeros((N, DX - H), _f32).at[:, :3].set(x)

    w1a = We1[:H]
    w1b = We1[H:2 * H]
    w1r = We1[2 * H]
    perm = np.empty((H,), dtype=np.int32)
    for p in range(H // 32):
        for i in range(16):
            perm[32 * p + 2 * i] = 32 * p + i
            perm[32 * p + 2 * i + 1] = 32 * p + 16 + i
    w1d = We1[2 * H + 1:][:, perm]
    We2 = We2[perm, :]

    a_ext, b_ext = _precompute(h, xp, w1a, w1b, be1.reshape(1, H))

    eat = edge_attr.T

    aggs = []
    for c in range(NCH):
        r2c = row2[c * NBLKC:(c + 1) * NBLKC]
        c2c = col2[c * NBLKC:(c + 1) * NBLKC]
        eatc = eat[:, c * EC:(c + 1) * EC]
        t = _sc_edge(a_ext, b_ext, r2c, c2c, w1r)
        ef2 = _mlp2(t, eatc, w1d, We2, be2.reshape(1, H))
        aggs.append(_sc_scatter(ef2, r2c))

    p = _node_pool(h, aggs[0], aggs[1], Wn1[:H], Wn1[H:], bn1.reshape(1, H),
                   Wn2, bn2.reshape(1, H), batch.reshape(N // 1000, 1, 1000))
    return p


# speedup vs baseline: 1.5296x; 1.5296x over previous
import jax
import jax.numpy as jnp
import numpy as np
from jax import lax
from jax.experimental import pallas as pl
from jax.experimental.pallas import tpu as pltpu
from jax.experimental.pallas import tpu_sc as plsc

N = 10000
E = 320000
H = 128
ED = 4
G = 64

DX = 144
EB = 128
NBLK = E // EB
NCH = 2
NBLKC = NBLK // NCH
EC = NBLKC * EB
NC = 2
NS = 16
NW = NC * NS
NBT = -(-NBLKC // NW)
NPAIR = (NBT + 1) // 2
NPT = N // NS
NPC = 125

_f32 = jnp.float32


def _silu(v):
    return v * jax.nn.sigmoid(v)


def _tile_blocks(wid):
    q, r = NBLKC // NW, NBLKC % NW
    start = q * wid + jnp.minimum(wid, r)
    nb = jnp.where(wid < r, q + 1, q)
    return start, nb


def _load_idx(src_hbm, start, nb, dst):
    @pl.when(nb == NBT)
    def _():
        pltpu.sync_copy(src_hbm.at[pl.ds(start, NBT)], dst)

    @pl.when(nb < NBT)
    def _():
        pltpu.sync_copy(src_hbm.at[pl.ds(start, NBT - 1)],
                        dst.at[pl.ds(0, NBT - 1)])


def _pre_body(h_ref, xp_ref, w1a_ref, w1b_ref, be1_ref, a_ref, b_ref):
    hb = h_ref[...]
    a = jnp.dot(hb, w1a_ref[...], preferred_element_type=_f32) + be1_ref[...]
    b = jnp.dot(hb, w1b_ref[...], preferred_element_type=_f32)
    a_ref[:, :H] = a
    a_ref[:, H:] = xp_ref[...]
    b_ref[:, :H] = b
    b_ref[:, H:] = xp_ref[...]


def _precompute(h, xp, w1a, w1b, be1):
    R = 2000
    grid = N // R
    return pl.pallas_call(
        _pre_body,
        grid=(grid,),
        in_specs=[
            pl.BlockSpec((R, H), lambda i: (i, 0)),
            pl.BlockSpec((R, DX - H), lambda i: (i, 0)),
            pl.BlockSpec((H, H), lambda i: (0, 0)),
            pl.BlockSpec((H, H), lambda i: (0, 0)),
            pl.BlockSpec((1, H), lambda i: (0, 0)),
        ],
        out_specs=[
            pl.BlockSpec((R, DX), lambda i: (i, 0)),
            pl.BlockSpec((R, DX), lambda i: (i, 0)),
        ],
        out_shape=[
            jax.ShapeDtypeStruct((N, DX), _f32),
            jax.ShapeDtypeStruct((N, DX), _f32),
        ],
    )(h, xp, w1a, w1b, be1)


def _sc_edge_body(a_hbm, b_hbm, row2_hbm, col2_hbm, w1r_hbm, t_hbm,
                  idxr_v, idxc_v, ar0, br0, ar1, br1, tb0, tb1, w1r_v,
                  sa0, sb0, sa1, sb1, st0, st1):
    wid = lax.axis_index("s") * NC + lax.axis_index("c")
    start, nb = _tile_blocks(wid)
    pltpu.sync_copy(w1r_hbm, w1r_v)
    _load_idx(row2_hbm, start, nb, idxr_v)
    _load_idx(col2_hbm, start, nb, idxc_v)

    def fetch(k, ar, br, sa, sb):
        pltpu.async_copy(a_hbm.at[idxr_v.at[k]], ar, sa)
        pltpu.async_copy(b_hbm.at[idxc_v.at[k]], br, sb)

    def compute(ar, br, tb):
        @plsc.parallel_loop(0, EB, unroll=2)
        def _edges(e):
            xa = ar[e, pl.ds(H, 16)]
            xb = br[e, pl.ds(H, 16)]
            d = xa - xb
            dd = d * d
            r = dd[0] + dd[1] + dd[2]
            for j in range(H // 16):
                s = pl.ds(16 * j, 16)
                tb[e, s] = ar[e, s] + br[e, s] + r * w1r_v[s]

    def phase(j, k, ar, br, tb, sa, sb, st):
        @pl.when(k < nb)
        def _():
            pltpu.make_async_copy(a_hbm.at[idxr_v.at[0]], ar, sa).wait()
            pltpu.make_async_copy(b_hbm.at[idxc_v.at[0]], br, sb).wait()

            @pl.when(j > 0)
            def _():
                pltpu.make_async_copy(tb, t_hbm.at[pl.ds(0, EB)], st).wait()

            compute(ar, br, tb)
            pltpu.async_copy(tb, t_hbm.at[pl.ds((start + k) * EB, EB)], st)

        @pl.when(k + 2 < nb)
        def _():
            fetch(k + 2, ar, br, sa, sb)

    fetch(0, ar0, br0, sa0, sb0)
    fetch(1, ar1, br1, sa1, sb1)

    @pl.loop(0, NPAIR)
    def _pairs(j):
        phase(j, 2 * j, ar0, br0, tb0, sa0, sb0, st0)
        phase(j, 2 * j + 1, ar1, br1, tb1, sa1, sb1, st1)

    pltpu.make_async_copy(tb0, t_hbm.at[pl.ds(0, EB)], st0).wait()
    pltpu.make_async_copy(tb1, t_hbm.at[pl.ds(0, EB)], st1).wait()


def _sc_edge(a_ext, b_ext, row2c, col2c, w1r):
    mesh = plsc.VectorSubcoreMesh(core_axis_name="c", subcore_axis_name="s")
    fn = pl.kernel(
        _sc_edge_body,
        out_type=jax.ShapeDtypeStruct((EC, H), _f32),
        mesh=mesh,
        compiler_params=pltpu.CompilerParams(use_tc_tiling_on_sc=False),
        scratch_types=[
            pltpu.VMEM((NBT, EB), jnp.int32),
            pltpu.VMEM((NBT, EB), jnp.int32),
            pltpu.VMEM((EB, DX), _f32),
            pltpu.VMEM((EB, DX), _f32),
            pltpu.VMEM((EB, DX), _f32),
            pltpu.VMEM((EB, DX), _f32),
            pltpu.VMEM((EB, H), _f32),
            pltpu.VMEM((EB, H), _f32),
            pltpu.VMEM((H,), _f32),
            pltpu.SemaphoreType.DMA,
            pltpu.SemaphoreType.DMA,
            pltpu.SemaphoreType.DMA,
            pltpu.SemaphoreType.DMA,
            pltpu.SemaphoreType.DMA,
            pltpu.SemaphoreType.DMA,
        ],
    )
    return fn(a_ext, b_ext, row2c, col2c, w1r)


def _mlp2_body(t_ref, eat_ref, w1d_ref, we2_ref, be2_ref, o_ref):
    ea_term = lax.dot_general(eat_ref[...], w1d_ref[...],
                              (((0,), (0,)), ((), ())),
                              preferred_element_type=_f32)
    ef = _silu(t_ref[...] + ea_term)
    o_ref[...] = _silu(jnp.dot(ef, we2_ref[...],
                               preferred_element_type=_f32) + be2_ref[...])


def _mlp2(t, eat, w1d, we2, be2):
    R = 1280
    grid = EC // R
    return pl.pallas_call(
        _mlp2_body,
        grid=(grid,),
        in_specs=[
            pl.BlockSpec((R, H), lambda i: (i, 0)),
            pl.BlockSpec((ED, R), lambda i: (0, i)),
            pl.BlockSpec((ED, H), lambda i: (0, 0)),
            pl.BlockSpec((H, H), lambda i: (0, 0)),
            pl.BlockSpec((1, H), lambda i: (0, 0)),
        ],
        out_specs=pl.BlockSpec((R, H), lambda i: (i, 0)),
        out_shape=jax.ShapeDtypeStruct((EC, H), _f32),
    )(t, eat, w1d, we2, be2)


def _sc_scatter_body(ef2_hbm, row2_hbm, agg_hbm,
                     idx_v, fb0, fb1, acc_sh, sf0, sf1, ss0, ss1):
    cid = lax.axis_index("c")
    sid = lax.axis_index("s")
    wid = sid * NC + cid
    start, nb = _tile_blocks(wid)
    _load_idx(row2_hbm, start, nb, idx_v)

    zero16 = jnp.zeros((16,), _f32)

    @pl.loop(0, NPC)
    def _zrow(e):
        for j in range(H // 16):
            fb0[e, pl.ds(16 * j, 16)] = zero16
    for i in range(NPT // NPC):
        pltpu.sync_copy(fb0.at[pl.ds(0, NPC)],
                        acc_sh.at[pl.ds(sid * NPT + i * NPC, NPC)])
    plsc.subcore_barrier()

    def phase_fetch(j, k, fb, sf, ss):
        @pl.when(k < nb)
        def _():
            @pl.when(j > 0)
            def _():
                pltpu.make_async_copy(fb, acc_sh.at[idx_v.at[0]], ss).wait()
            pltpu.async_copy(ef2_hbm.at[pl.ds((start + k) * EB, EB)], fb, sf)

    def phase_scatter(k, fb, sf, ss):
        @pl.when(k < nb)
        def _():
            pltpu.make_async_copy(
                ef2_hbm.at[pl.ds(0, EB)], fb, sf).wait()
            pltpu.async_copy(fb, acc_sh.at[idx_v.at[k]], ss, add=True)

    @pl.loop(0, NPAIR)
    def _pairs(j):
        phase_fetch(j, 2 * j, fb0, sf0, ss0)
        phase_fetch(j, 2 * j + 1, fb1, sf1, ss1)
        phase_scatter(2 * j, fb0, sf0, ss0)
        phase_scatter(2 * j + 1, fb1, sf1, ss1)

    pltpu.make_async_copy(fb0, acc_sh.at[idx_v.at[0]], ss0).wait()
    pltpu.make_async_copy(fb1, acc_sh.at[idx_v.at[0]], ss1).wait()

    plsc.subcore_barrier()
    for i in range(NPT // NPC):
        base = sid * NPT + i * NPC
        pltpu.sync_copy(acc_sh.at[pl.ds(base, NPC)], fb0.at[pl.ds(0, NPC)])
        pltpu.sync_copy(fb0.at[pl.ds(0, NPC)], agg_hbm.at[cid, pl.ds(base, NPC)])


def _sc_scatter(ef2c, row2c):
    mesh = plsc.VectorSubcoreMesh(core_axis_name="c", subcore_axis_name="s")
    fn = pl.kernel(
        _sc_scatter_body,
        out_type=jax.ShapeDtypeStruct((NC, N, H), _f32),
        mesh=mesh,
        compiler_params=pltpu.CompilerParams(use_tc_tiling_on_sc=False),
        scratch_types=[
            pltpu.VMEM((NBT, EB), jnp.int32),
            pltpu.VMEM((EB, H), _f32),
            pltpu.VMEM((EB, H), _f32),
            pltpu.VMEM_SHARED((N, H), _f32),
            pltpu.SemaphoreType.DMA,
            pltpu.SemaphoreType.DMA,
            pltpu.SemaphoreType.DMA,
            pltpu.SemaphoreType.DMA,
        ],
    )
    return fn(ef2c, row2c)


def _node_body(h_ref, agga_ref, aggb_ref, wn1a_ref, wn1b_ref, bn1_ref,
               wn2_ref, bn2_ref, batch_ref, p_ref, sums, cnt):
    i = pl.program_id(0)

    @pl.when(i == 0)
    def _init():
        sums[...] = jnp.zeros((G, H), _f32)
        cnt[...] = jnp.zeros((G, H), _f32)

    hb = h_ref[...]
    agg = (agga_ref[0] + agga_ref[1]) + (aggb_ref[0] + aggb_ref[1])
    pre = (jnp.dot(hb, wn1a_ref[...], preferred_element_type=_f32)
           + jnp.dot(agg, wn1b_ref[...], preferred_element_type=_f32)
           + bn1_ref[...])
    o = _silu(pre)
    o2 = jnp.dot(o, wn2_ref[...], preferred_element_type=_f32) + bn2_ref[...]
    h2 = hb + o2

    bvec = batch_ref[0, 0, :]
    R = h2.shape[0]
    oh = (bvec[:, None] == lax.broadcasted_iota(jnp.int32, (R, G), 1))
    ohf = oh.astype(_f32)
    sums[...] += lax.dot_general(ohf, h2, (((0,), (0,)), ((), ())),
                                 preferred_element_type=_f32)
    cnt[...] += jnp.sum(ohf, axis=0)[:, None]

    @pl.when(i == pl.num_programs(0) - 1)
    def _fin():
        p_ref[...] = sums[...] / jnp.maximum(cnt[...], 1.0)


def _node_pool(h, agga, aggb, wn1a, wn1b, bn1, wn2, bn2, batch3):
    R = 1000
    grid = N // R
    return pl.pallas_call(
        _node_body,
        grid=(grid,),
        in_specs=[
            pl.BlockSpec((R, H), lambda i: (i, 0)),
            pl.BlockSpec((NC, R, H), lambda i: (0, i, 0)),
            pl.BlockSpec((NC, R, H), lambda i: (0, i, 0)),
            pl.BlockSpec((H, H), lambda i: (0, 0)),
            pl.BlockSpec((H, H), lambda i: (0, 0)),
            pl.BlockSpec((1, H), lambda i: (0, 0)),
            pl.BlockSpec((H, H), lambda i: (0, 0)),
            pl.BlockSpec((1, H), lambda i: (0, 0)),
            pl.BlockSpec((1, 1, R), lambda i: (i, 0, 0)),
        ],
        out_specs=pl.BlockSpec((G, H), lambda i: (0, 0)),
        out_shape=jax.ShapeDtypeStruct((G, H), _f32),
        scratch_shapes=[
            pltpu.VMEM((G, H), _f32),
            pltpu.VMEM((G, H), _f32),
        ],
    )(h, agga, aggb, wn1a, wn1b, bn1, wn2, bn2, batch3)


def kernel(h, edge_index, x, edge_attr, batch, We1, be1, We2, be2,
           Wn1, bn1, Wn2, bn2):
    row2 = edge_index[0].reshape(NBLK, EB)
    col2 = edge_index[1].reshape(NBLK, EB)
    xp = jnp.zeros((N, DX - H), _f32).at[:, :3].set(x)

    w1a = We1[:H]
    w1b = We1[H:2 * H]
    w1r = We1[2 * H]
    w1d = We1[2 * H + 1:]

    a_ext, b_ext = _precompute(h, xp, w1a, w1b, be1.reshape(1, H))

    eat = edge_attr.T

    aggs = []
    for c in range(NCH):
        r2c = row2[c * NBLKC:(c + 1) * NBLKC]
        c2c = col2[c * NBLKC:(c + 1) * NBLKC]
        eatc = eat[:, c * EC:(c + 1) * EC]
        t = _sc_edge(a_ext, b_ext, r2c, c2c, w1r)
        ef2 = _mlp2(t, eatc, w1d, We2, be2.reshape(1, H))
        aggs.append(_sc_scatter(ef2, r2c))

    p = _node_pool(h, aggs[0], aggs[1], Wn1[:H], Wn1[H:], bn1.reshape(1, H),
                   Wn2, bn2.reshape(1, H), batch.reshape(N // 1000, 1, 1000))
    return p


# speedup vs baseline: 1.6926x; 1.1066x over previous
import jax
import jax.numpy as jnp
import numpy as np
from jax import lax
from jax.experimental import pallas as pl
from jax.experimental.pallas import tpu as pltpu
from jax.experimental.pallas import tpu_sc as plsc

N = 10000
E = 320000
H = 128
ED = 4
G = 64

DX = 144
EB = 128
NBLK = E // EB
NCH = 2
NBLKC = NBLK // NCH
EC = NBLKC * EB
NC = 2
NS = 16
NW = NC * NS
NBT = -(-NBLKC // NW)
NPAIR = (NBT + 1) // 2
NPT = N // NS
NPC = 125

_f32 = jnp.float32


def _silu(v):
    return v * jax.nn.sigmoid(v)


def _tile_blocks(wid):
    q, r = NBLKC // NW, NBLKC % NW
    start = q * wid + jnp.minimum(wid, r)
    nb = jnp.where(wid < r, q + 1, q)
    return start, nb


def _load_idx(src_hbm, start, nb, dst):
    @pl.when(nb == NBT)
    def _():
        pltpu.sync_copy(src_hbm.at[pl.ds(start, NBT)], dst)

    @pl.when(nb < NBT)
    def _():
        pltpu.sync_copy(src_hbm.at[pl.ds(start, NBT - 1)],
                        dst.at[pl.ds(0, NBT - 1)])


def _pre_body(h_ref, xp_ref, w1a_ref, w1b_ref, be1_ref, a_ref, b_ref):
    hb = h_ref[...]
    a = jnp.dot(hb, w1a_ref[...], preferred_element_type=_f32) + be1_ref[...]
    b = jnp.dot(hb, w1b_ref[...], preferred_element_type=_f32)
    a_ref[:, :H] = a
    a_ref[:, H:] = xp_ref[...]
    b_ref[:, :H] = b
    b_ref[:, H:] = xp_ref[...]


def _precompute(h, xp, w1a, w1b, be1):
    R = 2000
    grid = N // R
    return pl.pallas_call(
        _pre_body,
        grid=(grid,),
        in_specs=[
            pl.BlockSpec((R, H), lambda i: (i, 0)),
            pl.BlockSpec((R, DX - H), lambda i: (i, 0)),
            pl.BlockSpec((H, H), lambda i: (0, 0)),
            pl.BlockSpec((H, H), lambda i: (0, 0)),
            pl.BlockSpec((1, H), lambda i: (0, 0)),
        ],
        out_specs=[
            pl.BlockSpec((R, DX), lambda i: (i, 0)),
            pl.BlockSpec((R, DX), lambda i: (i, 0)),
        ],
        out_shape=[
            jax.ShapeDtypeStruct((N, DX), _f32),
            jax.ShapeDtypeStruct((N, DX), _f32),
        ],
    )(h, xp, w1a, w1b, be1)


def _sc_edge_body(a_hbm, b_hbm, row2_hbm, col2_hbm, w1r_hbm, t_hbm,
                  idxr_v, idxc_v, ar0, br0, ar1, br1, tb0, tb1, w1r_v,
                  sa0, sb0, sa1, sb1, st0, st1):
    wid = lax.axis_index("s") * NC + lax.axis_index("c")
    start, nb = _tile_blocks(wid)
    pltpu.sync_copy(w1r_hbm, w1r_v)
    _load_idx(row2_hbm, start, nb, idxr_v)
    _load_idx(col2_hbm, start, nb, idxc_v)

    def fetch(k, ar, br, sa, sb):
        pltpu.async_copy(a_hbm.at[idxr_v.at[k]], ar, sa)
        pltpu.async_copy(b_hbm.at[idxc_v.at[k]], br, sb)

    def compute(ar, br, tb):
        @plsc.parallel_loop(0, EB, unroll=2)
        def _edges(e):
            xa = ar[e, pl.ds(H, 16)]
            xb = br[e, pl.ds(H, 16)]
            d = xa - xb
            dd = d * d
            r = dd[0] + dd[1] + dd[2]
            for j in range(H // 16):
                s = pl.ds(16 * j, 16)
                tb[e, s] = ar[e, s] + br[e, s] + r * w1r_v[s]

    def phase(j, k, ar, br, tb, sa, sb, st):
        @pl.when(k < nb)
        def _():
            pltpu.make_async_copy(a_hbm.at[idxr_v.at[0]], ar, sa).wait()
            pltpu.make_async_copy(b_hbm.at[idxc_v.at[0]], br, sb).wait()

            @pl.when(j > 0)
            def _():
                pltpu.make_async_copy(tb, t_hbm.at[pl.ds(0, EB)], st).wait()

            compute(ar, br, tb)
            pltpu.async_copy(tb, t_hbm.at[pl.ds((start + k) * EB, EB)], st)

        @pl.when(k + 2 < nb)
        def _():
            fetch(k + 2, ar, br, sa, sb)

    fetch(0, ar0, br0, sa0, sb0)
    fetch(1, ar1, br1, sa1, sb1)

    @pl.loop(0, NPAIR)
    def _pairs(j):
        phase(j, 2 * j, ar0, br0, tb0, sa0, sb0, st0)
        phase(j, 2 * j + 1, ar1, br1, tb1, sa1, sb1, st1)

    pltpu.make_async_copy(tb0, t_hbm.at[pl.ds(0, EB)], st0).wait()
    pltpu.make_async_copy(tb1, t_hbm.at[pl.ds(0, EB)], st1).wait()


def _sc_edge(a_ext, b_ext, row2c, col2c, w1r):
    mesh = plsc.VectorSubcoreMesh(core_axis_name="c", subcore_axis_name="s")
    fn = pl.kernel(
        _sc_edge_body,
        out_type=jax.ShapeDtypeStruct((EC, H), _f32),
        mesh=mesh,
        compiler_params=pltpu.CompilerParams(use_tc_tiling_on_sc=False),
        scratch_types=[
            pltpu.VMEM((NBT, EB), jnp.int32),
            pltpu.VMEM((NBT, EB), jnp.int32),
            pltpu.VMEM((EB, DX), _f32),
            pltpu.VMEM((EB, DX), _f32),
            pltpu.VMEM((EB, DX), _f32),
            pltpu.VMEM((EB, DX), _f32),
            pltpu.VMEM((EB, H), _f32),
            pltpu.VMEM((EB, H), _f32),
            pltpu.VMEM((H,), _f32),
            pltpu.SemaphoreType.DMA,
            pltpu.SemaphoreType.DMA,
            pltpu.SemaphoreType.DMA,
            pltpu.SemaphoreType.DMA,
            pltpu.SemaphoreType.DMA,
            pltpu.SemaphoreType.DMA,
        ],
    )
    return fn(a_ext, b_ext, row2c, col2c, w1r)


def _mlp2_body(t_ref, eat_ref, w1d_ref, we2_ref, be2_ref, o_ref):
    ea_term = lax.dot_general(eat_ref[...], w1d_ref[...],
                              (((0,), (0,)), ((), ())),
                              preferred_element_type=_f32)
    ef = _silu(t_ref[...] + ea_term)
    o_ref[...] = _silu(jnp.dot(ef, we2_ref[...],
                               preferred_element_type=_f32) + be2_ref[...])


def _mlp2(t, eat, w1d, we2, be2):
    R = 3200
    grid = EC // R
    return pl.pallas_call(
        _mlp2_body,
        grid=(grid,),
        in_specs=[
            pl.BlockSpec((R, H), lambda i: (i, 0)),
            pl.BlockSpec((ED, R), lambda i: (0, i)),
            pl.BlockSpec((ED, H), lambda i: (0, 0)),
            pl.BlockSpec((H, H), lambda i: (0, 0)),
            pl.BlockSpec((1, H), lambda i: (0, 0)),
        ],
        out_specs=pl.BlockSpec((R, H), lambda i: (i, 0)),
        out_shape=jax.ShapeDtypeStruct((EC, H), _f32),
    )(t, eat, w1d, we2, be2)


def _sc_scatter_body(ef2_hbm, row2_hbm, agg_hbm,
                     idx_v, fb0, fb1, acc_sh, sf0, sf1, ss0, ss1):
    cid = lax.axis_index("c")
    sid = lax.axis_index("s")
    wid = sid * NC + cid
    start, nb = _tile_blocks(wid)
    _load_idx(row2_hbm, start, nb, idx_v)

    zero16 = jnp.zeros((16,), _f32)

    @pl.loop(0, NPC)
    def _zrow(e):
        for j in range(H // 16):
            fb0[e, pl.ds(16 * j, 16)] = zero16
    for i in range(NPT // NPC):
        pltpu.sync_copy(fb0.at[pl.ds(0, NPC)],
                        acc_sh.at[pl.ds(sid * NPT + i * NPC, NPC)])
    plsc.subcore_barrier()

    def phase_fetch(j, k, fb, sf, ss):
        @pl.when(k < nb)
        def _():
            @pl.when(j > 0)
            def _():
                pltpu.make_async_copy(fb, acc_sh.at[idx_v.at[0]], ss).wait()
            pltpu.async_copy(ef2_hbm.at[pl.ds((start + k) * EB, EB)], fb, sf)

    def phase_scatter(k, fb, sf, ss):
        @pl.when(k < nb)
        def _():
            pltpu.make_async_copy(
                ef2_hbm.at[pl.ds(0, EB)], fb, sf).wait()
            pltpu.async_copy(fb, acc_sh.at[idx_v.at[k]], ss, add=True)

    @pl.loop(0, NPAIR)
    def _pairs(j):
        phase_fetch(j, 2 * j, fb0, sf0, ss0)
        phase_fetch(j, 2 * j + 1, fb1, sf1, ss1)
        phase_scatter(2 * j, fb0, sf0, ss0)
        phase_scatter(2 * j + 1, fb1, sf1, ss1)

    pltpu.make_async_copy(fb0, acc_sh.at[idx_v.at[0]], ss0).wait()
    pltpu.make_async_copy(fb1, acc_sh.at[idx_v.at[0]], ss1).wait()

    plsc.subcore_barrier()
    for i in range(NPT // NPC):
        base = sid * NPT + i * NPC
        pltpu.sync_copy(acc_sh.at[pl.ds(base, NPC)], fb0.at[pl.ds(0, NPC)])
        pltpu.sync_copy(fb0.at[pl.ds(0, NPC)], agg_hbm.at[cid, pl.ds(base, NPC)])


def _sc_scatter(ef2c, row2c):
    mesh = plsc.VectorSubcoreMesh(core_axis_name="c", subcore_axis_name="s")
    fn = pl.kernel(
        _sc_scatter_body,
        out_type=jax.ShapeDtypeStruct((NC, N, H), _f32),
        mesh=mesh,
        compiler_params=pltpu.CompilerParams(use_tc_tiling_on_sc=False),
        scratch_types=[
            pltpu.VMEM((NBT, EB), jnp.int32),
            pltpu.VMEM((EB, H), _f32),
            pltpu.VMEM((EB, H), _f32),
            pltpu.VMEM_SHARED((N, H), _f32),
            pltpu.SemaphoreType.DMA,
            pltpu.SemaphoreType.DMA,
            pltpu.SemaphoreType.DMA,
            pltpu.SemaphoreType.DMA,
        ],
    )
    return fn(ef2c, row2c)


def _node_body(h_ref, agga_ref, aggb_ref, wn1a_ref, wn1b_ref, bn1_ref,
               wn2_ref, bn2_ref, batch_ref, p_ref, sums, cnt):
    i = pl.program_id(0)

    @pl.when(i == 0)
    def _init():
        sums[...] = jnp.zeros((G, H), _f32)
        cnt[...] = jnp.zeros((G, H), _f32)

    hb = h_ref[...]
    agg = (agga_ref[0] + agga_ref[1]) + (aggb_ref[0] + aggb_ref[1])
    pre = (jnp.dot(hb, wn1a_ref[...], preferred_element_type=_f32)
           + jnp.dot(agg, wn1b_ref[...], preferred_element_type=_f32)
           + bn1_ref[...])
    o = _silu(pre)
    o2 = jnp.dot(o, wn2_ref[...], preferred_element_type=_f32) + bn2_ref[...]
    h2 = hb + o2

    bvec = batch_ref[0, 0, :]
    R = h2.shape[0]
    oh = (bvec[:, None] == lax.broadcasted_iota(jnp.int32, (R, G), 1))
    ohf = oh.astype(_f32)
    sums[...] += lax.dot_general(ohf, h2, (((0,), (0,)), ((), ())),
                                 preferred_element_type=_f32)
    cnt[...] += jnp.sum(ohf, axis=0)[:, None]

    @pl.when(i == pl.num_programs(0) - 1)
    def _fin():
        p_ref[...] = sums[...] / jnp.maximum(cnt[...], 1.0)


def _node_pool(h, agga, aggb, wn1a, wn1b, bn1, wn2, bn2, batch3):
    R = 1000
    grid = N // R
    return pl.pallas_call(
        _node_body,
        grid=(grid,),
        in_specs=[
            pl.BlockSpec((R, H), lambda i: (i, 0)),
            pl.BlockSpec((NC, R, H), lambda i: (0, i, 0)),
            pl.BlockSpec((NC, R, H), lambda i: (0, i, 0)),
            pl.BlockSpec((H, H), lambda i: (0, 0)),
            pl.BlockSpec((H, H), lambda i: (0, 0)),
            pl.BlockSpec((1, H), lambda i: (0, 0)),
            pl.BlockSpec((H, H), lambda i: (0, 0)),
            pl.BlockSpec((1, H), lambda i: (0, 0)),
            pl.BlockSpec((1, 1, R), lambda i: (i, 0, 0)),
        ],
        out_specs=pl.BlockSpec((G, H), lambda i: (0, 0)),
        out_shape=jax.ShapeDtypeStruct((G, H), _f32),
        scratch_shapes=[
            pltpu.VMEM((G, H), _f32),
            pltpu.VMEM((G, H), _f32),
        ],
    )(h, agga, aggb, wn1a, wn1b, bn1, wn2, bn2, batch3)


def kernel(h, edge_index, x, edge_attr, batch, We1, be1, We2, be2,
           Wn1, bn1, Wn2, bn2):
    row2 = edge_index[0].reshape(NBLK, EB)
    col2 = edge_index[1].reshape(NBLK, EB)
    xp = jnp.zeros((N, DX - H), _f32).at[:, :3].set(x)

    w1a = We1[:H]
    w1b = We1[H:2 * H]
    w1r = We1[2 * H]
    w1d = We1[2 * H + 1:]

    a_ext, b_ext = _precompute(h, xp, w1a, w1b, be1.reshape(1, H))

    eat = edge_attr.T

    aggs = []
    for c in range(NCH):
        r2c = row2[c * NBLKC:(c + 1) * NBLKC]
        c2c = col2[c * NBLKC:(c + 1) * NBLKC]
        eatc = eat[:, c * EC:(c + 1) * EC]
        t = _sc_edge(a_ext, b_ext, r2c, c2c, w1r)
        ef2 = _mlp2(t, eatc, w1d, We2, be2.reshape(1, H))
        aggs.append(_sc_scatter(ef2, r2c))

    p = _node_pool(h, aggs[0], aggs[1], Wn1[:H], Wn1[H:], bn1.reshape(1, H),
                   Wn2, bn2.reshape(1, H), batch.reshape(N // 1000, 1, 1000))
    return p


# speedup vs baseline: 1.6973x; 1.0028x over previous
import jax
import jax.numpy as jnp
import numpy as np
from jax import lax
from jax.experimental import pallas as pl
from jax.experimental.pallas import tpu as pltpu
from jax.experimental.pallas import tpu_sc as plsc

N = 10000
E = 320000
H = 128
ED = 4
G = 64

DX = 144
EB = 128
NBLK = E // EB
NCH = 2
NBLKC = NBLK // NCH
EC = NBLKC * EB
NC = 2
NS = 16
NW = NC * NS
NBT = -(-NBLKC // NW)
NPAIR = (NBT + 1) // 2
NPT = N // NS
NPC = 125

_f32 = jnp.float32


def _silu(v):
    return v * jax.nn.sigmoid(v)


def _tile_blocks(wid):
    q, r = NBLKC // NW, NBLKC % NW
    start = q * wid + jnp.minimum(wid, r)
    nb = jnp.where(wid < r, q + 1, q)
    return start, nb


def _load_idx(src_hbm, start, nb, dst):
    @pl.when(nb == NBT)
    def _():
        pltpu.sync_copy(src_hbm.at[pl.ds(start, NBT)], dst)

    @pl.when(nb < NBT)
    def _():
        pltpu.sync_copy(src_hbm.at[pl.ds(start, NBT - 1)],
                        dst.at[pl.ds(0, NBT - 1)])


def _pre_body(h_ref, xp_ref, w1a_ref, w1b_ref, be1_ref, a_ref, b_ref):
    hb = h_ref[...]
    a = jnp.dot(hb, w1a_ref[...], preferred_element_type=_f32) + be1_ref[...]
    b = jnp.dot(hb, w1b_ref[...], preferred_element_type=_f32)
    a_ref[:, :H] = a
    a_ref[:, H:] = xp_ref[...]
    b_ref[:, :H] = b
    b_ref[:, H:] = xp_ref[...]


def _precompute(h, xp, w1a, w1b, be1):
    R = 2000
    grid = N // R
    return pl.pallas_call(
        _pre_body,
        grid=(grid,),
        in_specs=[
            pl.BlockSpec((R, H), lambda i: (i, 0)),
            pl.BlockSpec((R, DX - H), lambda i: (i, 0)),
            pl.BlockSpec((H, H), lambda i: (0, 0)),
            pl.BlockSpec((H, H), lambda i: (0, 0)),
            pl.BlockSpec((1, H), lambda i: (0, 0)),
        ],
        out_specs=[
            pl.BlockSpec((R, DX), lambda i: (i, 0)),
            pl.BlockSpec((R, DX), lambda i: (i, 0)),
        ],
        out_shape=[
            jax.ShapeDtypeStruct((N, DX), _f32),
            jax.ShapeDtypeStruct((N, DX), _f32),
        ],
    )(h, xp, w1a, w1b, be1)


def _sc_edge_body(a_hbm, b_hbm, row2_hbm, col2_hbm, w1r_hbm, t_hbm,
                  idxr_v, idxc_v, ar0, br0, ar1, br1, tb0, tb1, w1r_v,
                  sa0, sb0, sa1, sb1, st0, st1):
    wid = lax.axis_index("s") * NC + lax.axis_index("c")
    start, nb = _tile_blocks(wid)
    pltpu.sync_copy(w1r_hbm, w1r_v)
    _load_idx(row2_hbm, start, nb, idxr_v)
    _load_idx(col2_hbm, start, nb, idxc_v)

    def fetch(k, ar, br, sa, sb):
        pltpu.async_copy(a_hbm.at[idxr_v.at[k]], ar, sa)
        pltpu.async_copy(b_hbm.at[idxc_v.at[k]], br, sb)

    def compute(ar, br, tb):
        @plsc.parallel_loop(0, EB, unroll=2)
        def _edges(e):
            xa = ar[e, pl.ds(H, 16)]
            xb = br[e, pl.ds(H, 16)]
            d = xa - xb
            dd = d * d
            r = dd[0] + dd[1] + dd[2]
            for j in range(H // 16):
                s = pl.ds(16 * j, 16)
                tb[e, s] = ar[e, s] + br[e, s] + r * w1r_v[s]

    def phase(j, k, ar, br, tb, sa, sb, st):
        @pl.when(k < nb)
        def _():
            pltpu.make_async_copy(a_hbm.at[idxr_v.at[0]], ar, sa).wait()
            pltpu.make_async_copy(b_hbm.at[idxc_v.at[0]], br, sb).wait()

            @pl.when(j > 0)
            def _():
                pltpu.make_async_copy(tb, t_hbm.at[pl.ds(0, EB)], st).wait()

            compute(ar, br, tb)
            pltpu.async_copy(tb, t_hbm.at[pl.ds((start + k) * EB, EB)], st)

        @pl.when(k + 2 < nb)
        def _():
            fetch(k + 2, ar, br, sa, sb)

    fetch(0, ar0, br0, sa0, sb0)
    fetch(1, ar1, br1, sa1, sb1)

    @pl.loop(0, NPAIR)
    def _pairs(j):
        phase(j, 2 * j, ar0, br0, tb0, sa0, sb0, st0)
        phase(j, 2 * j + 1, ar1, br1, tb1, sa1, sb1, st1)

    pltpu.make_async_copy(tb0, t_hbm.at[pl.ds(0, EB)], st0).wait()
    pltpu.make_async_copy(tb1, t_hbm.at[pl.ds(0, EB)], st1).wait()


def _sc_edge(a_ext, b_ext, row2c, col2c, w1r):
    mesh = plsc.VectorSubcoreMesh(core_axis_name="c", subcore_axis_name="s")
    fn = pl.kernel(
        _sc_edge_body,
        out_type=jax.ShapeDtypeStruct((EC, H), _f32),
        mesh=mesh,
        compiler_params=pltpu.CompilerParams(use_tc_tiling_on_sc=False),
        scratch_types=[
            pltpu.VMEM((NBT, EB), jnp.int32),
            pltpu.VMEM((NBT, EB), jnp.int32),
            pltpu.VMEM((EB, DX), _f32),
            pltpu.VMEM((EB, DX), _f32),
            pltpu.VMEM((EB, DX), _f32),
            pltpu.VMEM((EB, DX), _f32),
            pltpu.VMEM((EB, H), _f32),
            pltpu.VMEM((EB, H), _f32),
            pltpu.VMEM((H,), _f32),
            pltpu.SemaphoreType.DMA,
            pltpu.SemaphoreType.DMA,
            pltpu.SemaphoreType.DMA,
            pltpu.SemaphoreType.DMA,
            pltpu.SemaphoreType.DMA,
            pltpu.SemaphoreType.DMA,
        ],
    )
    return fn(a_ext, b_ext, row2c, col2c, w1r)


def _mlp2_body(t_ref, eat_ref, w1d_ref, we2_ref, be2_ref, o_ref):
    ea_term = lax.dot_general(eat_ref[...], w1d_ref[...],
                              (((0,), (0,)), ((), ())),
                              preferred_element_type=_f32)
    ef = _silu(t_ref[...] + ea_term)
    o_ref[...] = _silu(jnp.dot(ef, we2_ref[...],
                               preferred_element_type=_f32) + be2_ref[...])


def _mlp2(t, eat, w1d, we2, be2):
    R = 6400
    grid = EC // R
    return pl.pallas_call(
        _mlp2_body,
        grid=(grid,),
        in_specs=[
            pl.BlockSpec((R, H), lambda i: (i, 0)),
            pl.BlockSpec((ED, R), lambda i: (0, i)),
            pl.BlockSpec((ED, H), lambda i: (0, 0)),
            pl.BlockSpec((H, H), lambda i: (0, 0)),
            pl.BlockSpec((1, H), lambda i: (0, 0)),
        ],
        out_specs=pl.BlockSpec((R, H), lambda i: (i, 0)),
        out_shape=jax.ShapeDtypeStruct((EC, H), _f32),
    )(t, eat, w1d, we2, be2)


def _sc_scatter_body(ef2_hbm, row2_hbm, agg_hbm,
                     idx_v, fb0, fb1, acc_sh, sf0, sf1, ss0, ss1):
    cid = lax.axis_index("c")
    sid = lax.axis_index("s")
    wid = sid * NC + cid
    start, nb = _tile_blocks(wid)
    _load_idx(row2_hbm, start, nb, idx_v)

    zero16 = jnp.zeros((16,), _f32)

    @pl.loop(0, NPC)
    def _zrow(e):
        for j in range(H // 16):
            fb0[e, pl.ds(16 * j, 16)] = zero16
    for i in range(NPT // NPC):
        pltpu.sync_copy(fb0.at[pl.ds(0, NPC)],
                        acc_sh.at[pl.ds(sid * NPT + i * NPC, NPC)])
    plsc.subcore_barrier()

    def phase_fetch(j, k, fb, sf, ss):
        @pl.when(k < nb)
        def _():
            @pl.when(j > 0)
            def _():
                pltpu.make_async_copy(fb, acc_sh.at[idx_v.at[0]], ss).wait()
            pltpu.async_copy(ef2_hbm.at[pl.ds((start + k) * EB, EB)], fb, sf)

    def phase_scatter(k, fb, sf, ss):
        @pl.when(k < nb)
        def _():
            pltpu.make_async_copy(
                ef2_hbm.at[pl.ds(0, EB)], fb, sf).wait()
            pltpu.async_copy(fb, acc_sh.at[idx_v.at[k]], ss, add=True)

    @pl.loop(0, NPAIR)
    def _pairs(j):
        phase_fetch(j, 2 * j, fb0, sf0, ss0)
        phase_fetch(j, 2 * j + 1, fb1, sf1, ss1)
        phase_scatter(2 * j, fb0, sf0, ss0)
        phase_scatter(2 * j + 1, fb1, sf1, ss1)

    pltpu.make_async_copy(fb0, acc_sh.at[idx_v.at[0]], ss0).wait()
    pltpu.make_async_copy(fb1, acc_sh.at[idx_v.at[0]], ss1).wait()

    plsc.subcore_barrier()
    for i in range(NPT // NPC):
        base = sid * NPT + i * NPC
        pltpu.sync_copy(acc_sh.at[pl.ds(base, NPC)], fb0.at[pl.ds(0, NPC)])
        pltpu.sync_copy(fb0.at[pl.ds(0, NPC)], agg_hbm.at[cid, pl.ds(base, NPC)])


def _sc_scatter(ef2c, row2c):
    mesh = plsc.VectorSubcoreMesh(core_axis_name="c", subcore_axis_name="s")
    fn = pl.kernel(
        _sc_scatter_body,
        out_type=jax.ShapeDtypeStruct((NC, N, H), _f32),
        mesh=mesh,
        compiler_params=pltpu.CompilerParams(use_tc_tiling_on_sc=False),
        scratch_types=[
            pltpu.VMEM((NBT, EB), jnp.int32),
            pltpu.VMEM((EB, H), _f32),
            pltpu.VMEM((EB, H), _f32),
            pltpu.VMEM_SHARED((N, H), _f32),
            pltpu.SemaphoreType.DMA,
            pltpu.SemaphoreType.DMA,
            pltpu.SemaphoreType.DMA,
            pltpu.SemaphoreType.DMA,
        ],
    )
    return fn(ef2c, row2c)


def _node_body(h_ref, agga_ref, aggb_ref, wn1a_ref, wn1b_ref, bn1_ref,
               wn2_ref, bn2_ref, batch_ref, p_ref, sums, cnt):
    i = pl.program_id(0)

    @pl.when(i == 0)
    def _init():
        sums[...] = jnp.zeros((G, H), _f32)
        cnt[...] = jnp.zeros((G, H), _f32)

    hb = h_ref[...]
    agg = (agga_ref[0] + agga_ref[1]) + (aggb_ref[0] + aggb_ref[1])
    pre = (jnp.dot(hb, wn1a_ref[...], preferred_element_type=_f32)
           + jnp.dot(agg, wn1b_ref[...], preferred_element_type=_f32)
           + bn1_ref[...])
    o = _silu(pre)
    o2 = jnp.dot(o, wn2_ref[...], preferred_element_type=_f32) + bn2_ref[...]
    h2 = hb + o2

    bvec = batch_ref[0, 0, :]
    R = h2.shape[0]
    oh = (bvec[:, None] == lax.broadcasted_iota(jnp.int32, (R, G), 1))
    ohf = oh.astype(_f32)
    sums[...] += lax.dot_general(ohf, h2, (((0,), (0,)), ((), ())),
                                 preferred_element_type=_f32)
    cnt[...] += jnp.sum(ohf, axis=0)[:, None]

    @pl.when(i == pl.num_programs(0) - 1)
    def _fin():
        p_ref[...] = sums[...] / jnp.maximum(cnt[...], 1.0)


def _node_pool(h, agga, aggb, wn1a, wn1b, bn1, wn2, bn2, batch3):
    R = 2000
    grid = N // R
    return pl.pallas_call(
        _node_body,
        grid=(grid,),
        in_specs=[
            pl.BlockSpec((R, H), lambda i: (i, 0)),
            pl.BlockSpec((NC, R, H), lambda i: (0, i, 0)),
            pl.BlockSpec((NC, R, H), lambda i: (0, i, 0)),
            pl.BlockSpec((H, H), lambda i: (0, 0)),
            pl.BlockSpec((H, H), lambda i: (0, 0)),
            pl.BlockSpec((1, H), lambda i: (0, 0)),
            pl.BlockSpec((H, H), lambda i: (0, 0)),
            pl.BlockSpec((1, H), lambda i: (0, 0)),
            pl.BlockSpec((1, 1, R), lambda i: (i, 0, 0)),
        ],
        out_specs=pl.BlockSpec((G, H), lambda i: (0, 0)),
        out_shape=jax.ShapeDtypeStruct((G, H), _f32),
        scratch_shapes=[
            pltpu.VMEM((G, H), _f32),
            pltpu.VMEM((G, H), _f32),
        ],
    )(h, agga, aggb, wn1a, wn1b, bn1, wn2, bn2, batch3)


def kernel(h, edge_index, x, edge_attr, batch, We1, be1, We2, be2,
           Wn1, bn1, Wn2, bn2):
    row2 = edge_index[0].reshape(NBLK, EB)
    col2 = edge_index[1].reshape(NBLK, EB)
    xp = jnp.zeros((N, DX - H), _f32).at[:, :3].set(x)

    w1a = We1[:H]
    w1b = We1[H:2 * H]
    w1r = We1[2 * H]
    w1d = We1[2 * H + 1:]

    a_ext, b_ext = _precompute(h, xp, w1a, w1b, be1.reshape(1, H))

    eat = edge_attr.T

    aggs = []
    for c in range(NCH):
        r2c = row2[c * NBLKC:(c + 1) * NBLKC]
        c2c = col2[c * NBLKC:(c + 1) * NBLKC]
        eatc = eat[:, c * EC:(c + 1) * EC]
        t = _sc_edge(a_ext, b_ext, r2c, c2c, w1r)
        ef2 = _mlp2(t, eatc, w1d, We2, be2.reshape(1, H))
        aggs.append(_sc_scatter(ef2, r2c))

    p = _node_pool(h, aggs[0], aggs[1], Wn1[:H], Wn1[H:], bn1.reshape(1, H),
                   Wn2, bn2.reshape(1, H), batch.reshape(N // 2000, 1, 2000))
    return p
